# baseline (speedup 1.0000x reference)
import functools

import jax
import jax.numpy as jnp
from jax import lax
from jax.experimental import pallas as pl
from jax.experimental.pallas import tpu as pltpu
from jax.experimental.pallas import tpu_sc as plsc

N = 10000
E = 320000
LATENT = 128
STEPS = 2

NC = 2
NS = 16
NW = NC * NS
EH = E // 2
N_PAD = 10240
ROWS_PER_SUB = N_PAD // NS

_mesh = plsc.VectorSubcoreMesh(core_axis_name="c", subcore_axis_name="s")


def _make_gather(eoff, esize, ch, NSLOT=5):
    per_w = esize // NW
    nch = per_w // ch
    assert per_w % ch == 0 and ch % 8 == 0 and nch >= 2 * NSLOT

    @functools.partial(
        pl.kernel,
        out_type=jax.ShapeDtypeStruct((esize, LATENT), jnp.float32),
        mesh=_mesh,
        scratch_types=[
            pltpu.VMEM((per_w,), jnp.int32),
            pltpu.VMEM((per_w,), jnp.int32),
        ] + [pltpu.VMEM((ch, LATENT), jnp.float32)] * NSLOT
          + [pltpu.SemaphoreType.DMA] * (2 * NSLOT),
    )
    def _gather(ub_hbm, uc_hbm, s_hbm, r_hbm, g_hbm, sidx, ridx, *bufs):
        row = bufs[0:NSLOT]
        gsem = bufs[NSLOT:2 * NSLOT]
        wsem = bufs[2 * NSLOT:3 * NSLOT]
        cid = lax.axis_index("c")
        sid = lax.axis_index("s")
        wid = sid * NC + cid
        base = wid * per_w

        pltpu.sync_copy(s_hbm.at[pl.ds(eoff + base, per_w)], sidx)
        pltpu.sync_copy(r_hbm.at[pl.ds(eoff + base, per_w)], ridx)

        def issue_g1(c, k):
            off = c * ch
            pltpu.async_copy(ub_hbm.at[sidx.at[pl.ds(off, ch)]], row[k],
                             gsem[k])

        def issue_g2(c, k):
            off = c * ch
            pltpu.async_copy(uc_hbm.at[ridx.at[pl.ds(off, ch)]], row[k],
                             gsem[k], add=True)

        def wait_g(k):
            pltpu.make_async_copy(ub_hbm.at[pl.ds(0, ch)], row[k],
                                  gsem[k]).wait()

        def issue_w(c, k):
            off = base + c * ch
            pltpu.async_copy(row[k], g_hbm.at[pl.ds(off, ch)], wsem[k])

        def wait_w(k):
            pltpu.make_async_copy(row[k], g_hbm.at[pl.ds(0, ch)],
                                  wsem[k]).wait()

        for k in range(NSLOT):
            issue_g1(k, k)

        def body(j, carry):
            for k in range(NSLOT):
                wait_g(k)
                issue_g2(NSLOT * j + k, k)
            for k in range(NSLOT):
                wait_g(k)
                issue_w(NSLOT * j + k, k)
            for k in range(NSLOT):
                wait_w(k)
                issue_g1(NSLOT * (j + 1) + k, k)
            return carry

        ngroups = nch // NSLOT - 1
        lax.fori_loop(0, ngroups, body, 0)

        tail0 = ngroups * NSLOT
        for k in range(NSLOT):
            wait_g(k)
            issue_g2(tail0 + k, k)
        for k in range(NSLOT):
            wait_g(k)
            issue_w(tail0 + k, k)
        for c in range(tail0 + NSLOT, nch):
            k = c % NSLOT
            wait_w(k)
            issue_g1(c, k)
            wait_g(k)
            issue_g2(c, k)
            wait_g(k)
            issue_w(c, k)
        for k in range(NSLOT):
            wait_w(k)

    return _gather


_gatherA = _make_gather(0, EH, 40, 8)
_gatherB = _make_gather(EH, EH, 40, 8)


def _make_segsum(eoff, esize, ch, NSLOT=4):
    per_w = esize // NW
    nch = per_w // ch
    assert per_w % ch == 0 and ch % 8 == 0 and nch >= 2 * NSLOT

    @functools.partial(
        pl.kernel,
        out_type=jax.ShapeDtypeStruct((NC, N_PAD, LATENT), jnp.float32),
        mesh=_mesh,
        scratch_types=[pltpu.VMEM((ch,), jnp.int32)] * NSLOT
          + [pltpu.VMEM((ch, LATENT), jnp.float32)] * NSLOT
          + [pltpu.SemaphoreType.DMA] * (2 * NSLOT)
          + [pltpu.VMEM_SHARED((N_PAD, LATENT), jnp.float32)],
    )
    def _segsum(e_hbm, ridx_hbm, zeros_hbm, out_hbm, *bufs):
        idx_v = bufs[0:NSLOT]
        rows = bufs[NSLOT:2 * NSLOT]
        lsem = bufs[2 * NSLOT:3 * NSLOT]
        ssem = bufs[3 * NSLOT:4 * NSLOT]
        acc_sh = bufs[4 * NSLOT]
        cid = lax.axis_index("c")
        sid = lax.axis_index("s")
        rbase = sid * ROWS_PER_SUB
        pltpu.sync_copy(zeros_hbm.at[pl.ds(rbase, ROWS_PER_SUB)],
                        acc_sh.at[pl.ds(rbase, ROWS_PER_SUB)])
        plsc.subcore_barrier()

        ebase = (cid * NS + sid) * per_w

        def issue_l(c, k):
            off = ebase + c * ch
            pltpu.async_copy(ridx_hbm.at[pl.ds(eoff + off, ch)], idx_v[k],
                             lsem[k])
            pltpu.async_copy(e_hbm.at[pl.ds(off, ch)], rows[k], lsem[k])

        def wait_l(k):
            pltpu.make_async_copy(ridx_hbm.at[pl.ds(0, ch)], idx_v[k],
                                  lsem[k]).wait()
            pltpu.make_async_copy(e_hbm.at[pl.ds(0, ch)], rows[k],
                                  lsem[k]).wait()

        def scat(k):
            pltpu.async_copy(rows[k], acc_sh.at[idx_v[k]], ssem[k], add=True)

        def wait_s(k):
            pltpu.make_async_copy(rows[k], acc_sh.at[pl.ds(0, ch)],
                                  ssem[k]).wait()

        for k in range(NSLOT):
            issue_l(k, k)

        def body(j, carry):
            ch0 = NSLOT + NSLOT * j
            for k in range(NSLOT):
                wait_l(k)
                scat(k)
            for k in range(NSLOT):
                wait_s(k)
                issue_l(ch0 + k, k)
            return carry

        ngroups = (nch - NSLOT) // NSLOT
        lax.fori_loop(0, ngroups, body, 0)
        for c in range(NSLOT + ngroups * NSLOT, nch):
            k = c % NSLOT
            wait_l(k)
            scat(k)
            wait_s(k)
            issue_l(c, k)
        for k in range(NSLOT):
            wait_l(k)
            scat(k)
            wait_s(k)

        plsc.subcore_barrier()
        pltpu.sync_copy(acc_sh.at[pl.ds(rbase, ROWS_PER_SUB)],
                        out_hbm.at[cid, pl.ds(rbase, ROWS_PER_SUB)])

    return _segsum


_segsumA = _make_segsum(0, EH, 40, 6)
_segsumB = _make_segsum(EH, EH, 40, 6)


def _ln(h, g, b):
    mu = jnp.mean(h, axis=-1, keepdims=True)
    var = jnp.mean((h - mu) * (h - mu), axis=-1, keepdims=True)
    return (h - mu) * lax.rsqrt(var + 1e-5) * g + b


def _dot(a, w):
    return jnp.dot(a, w, preferred_element_type=jnp.float32)


def _enc_body(x_ref, w1_ref, b1_ref, w2_ref, b2_ref, g_ref, b_ref, o_ref):
    h = jnp.maximum(_dot(x_ref[...], w1_ref[...]) + b1_ref[...], 0.0)
    h = _dot(h.astype(jnp.bfloat16), w2_ref[...]) + b2_ref[...]
    o_ref[...] = _ln(h, g_ref[...], b_ref[...])


def _encn_body(x_ref, w1_ref, b1_ref, w2_ref, b2_ref, g_ref, b_ref,
               wb_ref, wc_ref, o_ref, ub_ref, uc_ref):
    h = jnp.maximum(_dot(x_ref[...], w1_ref[...]) + b1_ref[...], 0.0)
    h = _dot(h, w2_ref[...]) + b2_ref[...]
    v = _ln(h, g_ref[...], b_ref[...])
    o_ref[...] = v
    ub_ref[...] = _dot(v, wb_ref[...])
    uc_ref[...] = _dot(v, wc_ref[...])


def _edge_body(e_ref, g_in_ref, w1a_ref, b1_ref,
               w2_ref, b2_ref, g_ref, b_ref, o_ref):
    e = e_ref[...]
    h = _dot(e.astype(jnp.bfloat16), w1a_ref[...]) + g_in_ref[...] + b1_ref[...]
    h = jnp.maximum(h, 0.0)
    h = _dot(h.astype(jnp.bfloat16), w2_ref[...]) + b2_ref[...]
    o_ref[...] = e + _ln(h, g_ref[...], b_ref[...])


def _node_body(emit_u, v_ref, aggA_ref, aggB_ref, wa_ref, wb_ref, b1_ref,
               w2_ref, b2_ref, g_ref, b_ref, *rest):
    v = v_ref[...]
    a = (aggA_ref[0] + aggA_ref[1]) + (aggB_ref[0] + aggB_ref[1])
    h = _dot(v, wa_ref[...]) + _dot(a, wb_ref[...]) + b1_ref[...]
    h = jnp.maximum(h, 0.0)
    h = _dot(h, w2_ref[...]) + b2_ref[...]
    vn = v + _ln(h, g_ref[...], b_ref[...])
    if emit_u:
        nwb_ref, nwc_ref, o_ref, ub_ref, uc_ref = rest
        o_ref[...] = vn
        ub_ref[...] = _dot(vn, nwb_ref[...])
        uc_ref[...] = _dot(vn, nwc_ref[...])
    else:
        dw1_ref, db1_ref, dw2_ref, db2_ref, o_ref = rest
        hd = jnp.maximum(_dot(vn, dw1_ref[...]) + db1_ref[...], 0.0)
        o_ref[...] = _dot(hd, dw2_ref[...]) + db2_ref[...]


def _dec_body(v_ref, w1_ref, b1_ref, w2_ref, b2_ref, o_ref):
    h = jnp.maximum(_dot(v_ref[...], w1_ref[...]) + b1_ref[...], 0.0)
    o_ref[...] = _dot(h, w2_ref[...]) + b2_ref[...]


def _full(shape):
    return pl.BlockSpec(shape, lambda i: (0,) * len(shape))


def _rows(bsize, width):
    return pl.BlockSpec((bsize, width), lambda i: (i, 0))


def _enc_call(x, w1, b1, w2, b2, g, b, bsize, off_rows=0, m_out=None):
    m, f = x.shape
    m_out = m - off_rows if m_out is None else m_out
    off_b = off_rows // bsize
    in_spec = pl.BlockSpec((bsize, f), lambda i: (i + off_b, 0))
    return pl.pallas_call(
        _enc_body,
        grid=(m_out // bsize,),
        in_specs=[in_spec, _full(w1.shape), _full((1, LATENT)),
                  _full(w2.shape), _full((1, LATENT)), _full((1, LATENT)),
                  _full((1, LATENT))],
        out_specs=_rows(bsize, LATENT),
        out_shape=jax.ShapeDtypeStruct((m_out, LATENT), jnp.float32),
    )(x, w1, b1.reshape(1, -1), w2, b2.reshape(1, -1), g.reshape(1, -1),
      b.reshape(1, -1))


def _encn_call(x, w1, b1, w2, b2, g, b, wb, wc, bsize):
    m, f = x.shape
    return pl.pallas_call(
        _encn_body,
        grid=(m // bsize,),
        in_specs=[_rows(bsize, f), _full(w1.shape), _full((1, LATENT)),
                  _full(w2.shape), _full((1, LATENT)), _full((1, LATENT)),
                  _full((1, LATENT)), _full((LATENT, LATENT)),
                  _full((LATENT, LATENT))],
        out_specs=[_rows(bsize, LATENT)] * 3,
        out_shape=[jax.ShapeDtypeStruct((m, LATENT), jnp.float32)] * 3,
    )(x, w1, b1.reshape(1, -1), w2, b2.reshape(1, -1), g.reshape(1, -1),
      b.reshape(1, -1), wb, wc)


def _edge_call(e, gin, w1a, b1, w2, b2, g, b, bsize):
    m = e.shape[0]
    wspec = _full((LATENT, LATENT))
    vec = _full((1, LATENT))
    return pl.pallas_call(
        _edge_body,
        grid=(m // bsize,),
        in_specs=[_rows(bsize, LATENT)] * 2 + [wspec, vec,
                                               wspec, vec, vec, vec],
        out_specs=_rows(bsize, LATENT),
        out_shape=jax.ShapeDtypeStruct((m, LATENT), jnp.float32),
    )(e, gin, w1a, b1.reshape(1, -1), w2, b2.reshape(1, -1),
      g.reshape(1, -1), b.reshape(1, -1))


def _node_call(v, aggA, aggB, w1, b1, w2, b2, g, b, nwb, nwc, dec, bsize):
    wa, wb = w1[:LATENT], w1[LATENT:]
    wspec = _full((LATENT, LATENT))
    vec = _full((1, LATENT))
    agg_spec = pl.BlockSpec((NC, bsize, LATENT), lambda i: (0, i, 0))
    emit_u = nwb is not None
    in_specs = [_rows(bsize, LATENT), agg_spec, agg_spec, wspec, wspec, vec,
                wspec, vec, vec, vec]
    args = [v, aggA, aggB, wa, wb, b1.reshape(1, -1), w2, b2.reshape(1, -1),
            g.reshape(1, -1), b.reshape(1, -1)]
    if emit_u:
        in_specs += [wspec, wspec]
        args += [nwb, nwc]
        out_specs = [_rows(bsize, LATENT)] * 3
        out_shape = [jax.ShapeDtypeStruct((N, LATENT), jnp.float32)] * 3
    else:
        dw1, db1, dw2, db2 = dec
        out_f = dw2.shape[1]
        in_specs += [wspec, vec, _full((LATENT, out_f)), _full((1, out_f))]
        args += [dw1, db1.reshape(1, -1), dw2, db2.reshape(1, -1)]
        out_specs = _rows(bsize, out_f)
        out_shape = jax.ShapeDtypeStruct((N, out_f), jnp.float32)
    return pl.pallas_call(
        functools.partial(_node_body, emit_u),
        grid=(N // bsize,),
        in_specs=in_specs,
        out_specs=out_specs,
        out_shape=out_shape,
    )(*args)


def _dec_call(v, w1, b1, w2, b2, bsize):
    out_f = w2.shape[1]
    return pl.pallas_call(
        _dec_body,
        grid=(N // bsize,),
        in_specs=[_rows(bsize, LATENT), _full((LATENT, LATENT)),
                  _full((1, LATENT)), _full((LATENT, out_f)),
                  _full((1, out_f))],
        out_specs=_rows(bsize, out_f),
        out_shape=jax.ShapeDtypeStruct((N, out_f), jnp.float32),
    )(v, w1, b1.reshape(1, -1), w2, b2.reshape(1, -1))


def kernel(node_features, edge_features, params, senders, receivers):
    p = params
    w1 = [p['pe_W1'][t] for t in range(STEPS)]
    w1a = [w[:LATENT] for w in w1]
    w1b = [w[LATENT:2 * LATENT] for w in w1]
    w1c = [w[2 * LATENT:] for w in w1]

    v, ub, uc = _encn_call(node_features, p['enc_n_W1'], p['enc_n_b1'],
                           p['enc_n_W2'], p['enc_n_b2'], p['enc_n_g'],
                           p['enc_n_b'], w1b[0], w1c[0], 2000)
    enc_e_args = (p['enc_e_W1'], p['enc_e_b1'],
                  p['enc_e_W2'].astype(jnp.bfloat16),
                  p['enc_e_b2'], p['enc_e_g'], p['enc_e_b'])
    eA = _enc_call(edge_features, *enc_e_args, 4000, 0, EH)
    eB = _enc_call(edge_features, *enc_e_args, 4000, EH, EH)

    zeros = jnp.zeros((N_PAD, LATENT), jnp.float32)
    bf = jnp.bfloat16
    dec = (p['dec_W1'], p['dec_b1'], p['dec_W2'], p['dec_b2'])

    out = None
    for t in range(STEPS):
        ginA = _gatherA(ub, uc, senders, receivers)
        ginB = _gatherB(ub, uc, senders, receivers)
        edge_args = (w1a[t].astype(bf), p['pe_b1'][t],
                     p['pe_W2'][t].astype(bf), p['pe_b2'][t],
                     p['pe_g'][t], p['pe_b'][t], 4000)
        eA = _edge_call(eA, ginA, *edge_args)
        aggA = _segsumA(eA, receivers, zeros)
        eB = _edge_call(eB, ginB, *edge_args)
        aggB = _segsumB(eB, receivers, zeros)
        last = t == STEPS - 1
        res = _node_call(v, aggA, aggB, p['pn_W1'][t], p['pn_b1'][t],
                         p['pn_W2'][t], p['pn_b2'][t], p['pn_g'][t],
                         p['pn_b'][t],
                         None if last else w1b[t + 1],
                         None if last else w1c[t + 1],
                         dec if last else None, 2000)
        if last:
            out = res
        else:
            v, ub, uc = res

    return out


# speedup vs baseline: 1.0143x; 1.0143x over previous
import functools

import jax
import jax.numpy as jnp
from jax import lax
from jax.experimental import pallas as pl
from jax.experimental.pallas import tpu as pltpu
from jax.experimental.pallas import tpu_sc as plsc

N = 10000
E = 320000
LATENT = 128
STEPS = 2

NC = 2
NS = 16
NW = NC * NS
EH = E // 2
N_PAD = 10240
ROWS_PER_SUB = N_PAD // NS

_mesh = plsc.VectorSubcoreMesh(core_axis_name="c", subcore_axis_name="s")


def _make_gather(eoff, esize, ch, NSLOT=5):
    per_w = esize // NW
    nch = per_w // ch
    assert per_w % ch == 0 and ch % 8 == 0 and nch >= 2 * NSLOT

    @functools.partial(
        pl.kernel,
        out_type=jax.ShapeDtypeStruct((esize, LATENT), jnp.float32),
        mesh=_mesh,
        scratch_types=[
            pltpu.VMEM((per_w,), jnp.int32),
            pltpu.VMEM((per_w,), jnp.int32),
        ] + [pltpu.VMEM((ch, LATENT), jnp.float32)] * NSLOT
          + [pltpu.SemaphoreType.DMA] * (2 * NSLOT),
    )
    def _gather(ub_hbm, uc_hbm, s_hbm, r_hbm, g_hbm, sidx, ridx, *bufs):
        row = bufs[0:NSLOT]
        gsem = bufs[NSLOT:2 * NSLOT]
        wsem = bufs[2 * NSLOT:3 * NSLOT]
        cid = lax.axis_index("c")
        sid = lax.axis_index("s")
        wid = sid * NC + cid
        base = wid * per_w

        pltpu.sync_copy(s_hbm.at[pl.ds(eoff + base, per_w)], sidx)
        pltpu.sync_copy(r_hbm.at[pl.ds(eoff + base, per_w)], ridx)

        def issue_g1(c, k):
            off = c * ch
            pltpu.async_copy(ub_hbm.at[sidx.at[pl.ds(off, ch)]], row[k],
                             gsem[k])

        def issue_g2(c, k):
            off = c * ch
            pltpu.async_copy(uc_hbm.at[ridx.at[pl.ds(off, ch)]], row[k],
                             gsem[k], add=True)

        def wait_g(k):
            pltpu.make_async_copy(ub_hbm.at[pl.ds(0, ch)], row[k],
                                  gsem[k]).wait()

        def issue_w(c, k):
            off = base + c * ch
            pltpu.async_copy(row[k], g_hbm.at[pl.ds(off, ch)], wsem[k])

        def wait_w(k):
            pltpu.make_async_copy(row[k], g_hbm.at[pl.ds(0, ch)],
                                  wsem[k]).wait()

        for k in range(NSLOT):
            issue_g1(k, k)

        def body(j, carry):
            for k in range(NSLOT):
                wait_g(k)
                issue_g2(NSLOT * j + k, k)
            for k in range(NSLOT):
                wait_g(k)
                issue_w(NSLOT * j + k, k)
            for k in range(NSLOT):
                wait_w(k)
                issue_g1(NSLOT * (j + 1) + k, k)
            return carry

        ngroups = nch // NSLOT - 1
        lax.fori_loop(0, ngroups, body, 0)

        tail0 = ngroups * NSLOT
        for k in range(NSLOT):
            wait_g(k)
            issue_g2(tail0 + k, k)
        for k in range(NSLOT):
            wait_g(k)
            issue_w(tail0 + k, k)
        for c in range(tail0 + NSLOT, nch):
            k = c % NSLOT
            wait_w(k)
            issue_g1(c, k)
            wait_g(k)
            issue_g2(c, k)
            wait_g(k)
            issue_w(c, k)
        for k in range(NSLOT):
            wait_w(k)

    return _gather


_gatherA = _make_gather(0, EH, 40, 12)
_gatherB = _make_gather(EH, EH, 40, 12)


def _make_segsum(eoff, esize, ch, NSLOT=4):
    per_w = esize // NW
    nch = per_w // ch
    assert per_w % ch == 0 and ch % 8 == 0 and nch >= 2 * NSLOT

    @functools.partial(
        pl.kernel,
        out_type=jax.ShapeDtypeStruct((NC, N_PAD, LATENT), jnp.float32),
        mesh=_mesh,
        scratch_types=[pltpu.VMEM((ch,), jnp.int32)] * NSLOT
          + [pltpu.VMEM((ch, LATENT), jnp.float32)] * NSLOT
          + [pltpu.SemaphoreType.DMA] * NSLOT
          + [pltpu.VMEM_SHARED((N_PAD, LATENT), jnp.float32)],
    )
    def _segsum(e_hbm, ridx_hbm, zeros_hbm, out_hbm, *bufs):
        idx_v = bufs[0:NSLOT]
        rows = bufs[NSLOT:2 * NSLOT]
        lsem = bufs[2 * NSLOT:3 * NSLOT]
        acc_sh = bufs[3 * NSLOT]
        cid = lax.axis_index("c")
        sid = lax.axis_index("s")
        rbase = sid * ROWS_PER_SUB
        pltpu.sync_copy(zeros_hbm.at[pl.ds(rbase, ROWS_PER_SUB)],
                        acc_sh.at[pl.ds(rbase, ROWS_PER_SUB)])
        plsc.subcore_barrier()

        ebase = (cid * NS + sid) * per_w

        def issue_l(c, k):
            off = ebase + c * ch
            pltpu.async_copy(ridx_hbm.at[pl.ds(eoff + off, ch)], idx_v[k],
                             lsem[k])
            pltpu.async_copy(e_hbm.at[pl.ds(off, ch)], rows[k], lsem[k])

        def wait_l(k):
            pltpu.make_async_copy(ridx_hbm.at[pl.ds(0, ch)], idx_v[k],
                                  lsem[k]).wait()
            pltpu.make_async_copy(e_hbm.at[pl.ds(0, ch)], rows[k],
                                  lsem[k]).wait()

        def scat(k):
            pltpu.sync_copy(rows[k], acc_sh.at[idx_v[k]], add=True)

        for k in range(NSLOT):
            issue_l(k, k)

        def body(j, carry):
            ch0 = NSLOT + NSLOT * j
            for k in range(NSLOT):
                wait_l(k)
                scat(k)
                issue_l(ch0 + k, k)
            return carry

        ngroups = (nch - NSLOT) // NSLOT
        lax.fori_loop(0, ngroups, body, 0)
        for c in range(NSLOT + ngroups * NSLOT, nch):
            k = c % NSLOT
            wait_l(k)
            scat(k)
            issue_l(c, k)
        for k in range(NSLOT):
            wait_l(k)
            scat(k)

        plsc.subcore_barrier()
        pltpu.sync_copy(acc_sh.at[pl.ds(rbase, ROWS_PER_SUB)],
                        out_hbm.at[cid, pl.ds(rbase, ROWS_PER_SUB)])

    return _segsum


_segsumA = _make_segsum(0, EH, 40, 4)
_segsumB = _make_segsum(EH, EH, 40, 4)


def _ln(h, g, b):
    mu = jnp.mean(h, axis=-1, keepdims=True)
    var = jnp.mean((h - mu) * (h - mu), axis=-1, keepdims=True)
    return (h - mu) * lax.rsqrt(var + 1e-5) * g + b


def _dot(a, w):
    return jnp.dot(a, w, preferred_element_type=jnp.float32)


def _enc_body(x_ref, w1_ref, b1_ref, w2_ref, b2_ref, g_ref, b_ref, o_ref):
    h = jnp.maximum(_dot(x_ref[...], w1_ref[...]) + b1_ref[...], 0.0)
    h = _dot(h.astype(jnp.bfloat16), w2_ref[...]) + b2_ref[...]
    o_ref[...] = _ln(h, g_ref[...], b_ref[...])


def _encn_body(x_ref, w1_ref, b1_ref, w2_ref, b2_ref, g_ref, b_ref,
               wb_ref, wc_ref, o_ref, ub_ref, uc_ref):
    h = jnp.maximum(_dot(x_ref[...], w1_ref[...]) + b1_ref[...], 0.0)
    h = _dot(h, w2_ref[...]) + b2_ref[...]
    v = _ln(h, g_ref[...], b_ref[...])
    o_ref[...] = v
    ub_ref[...] = _dot(v, wb_ref[...])
    uc_ref[...] = _dot(v, wc_ref[...])


def _edge_body(e_ref, g_in_ref, w1a_ref, b1_ref,
               w2_ref, b2_ref, g_ref, b_ref, o_ref):
    e = e_ref[...]
    h = _dot(e.astype(jnp.bfloat16), w1a_ref[...]) + g_in_ref[...] + b1_ref[...]
    h = jnp.maximum(h, 0.0)
    h = _dot(h.astype(jnp.bfloat16), w2_ref[...]) + b2_ref[...]
    o_ref[...] = e + _ln(h, g_ref[...], b_ref[...])


def _node_body(emit_u, v_ref, aggA_ref, aggB_ref, wa_ref, wb_ref, b1_ref,
               w2_ref, b2_ref, g_ref, b_ref, *rest):
    v = v_ref[...]
    a = (aggA_ref[0] + aggA_ref[1]) + (aggB_ref[0] + aggB_ref[1])
    h = _dot(v, wa_ref[...]) + _dot(a, wb_ref[...]) + b1_ref[...]
    h = jnp.maximum(h, 0.0)
    h = _dot(h, w2_ref[...]) + b2_ref[...]
    vn = v + _ln(h, g_ref[...], b_ref[...])
    if emit_u:
        nwb_ref, nwc_ref, o_ref, ub_ref, uc_ref = rest
        o_ref[...] = vn
        ub_ref[...] = _dot(vn, nwb_ref[...])
        uc_ref[...] = _dot(vn, nwc_ref[...])
    else:
        dw1_ref, db1_ref, dw2_ref, db2_ref, o_ref = rest
        hd = jnp.maximum(_dot(vn, dw1_ref[...]) + db1_ref[...], 0.0)
        o_ref[...] = _dot(hd, dw2_ref[...]) + db2_ref[...]


def _dec_body(v_ref, w1_ref, b1_ref, w2_ref, b2_ref, o_ref):
    h = jnp.maximum(_dot(v_ref[...], w1_ref[...]) + b1_ref[...], 0.0)
    o_ref[...] = _dot(h, w2_ref[...]) + b2_ref[...]


def _full(shape):
    return pl.BlockSpec(shape, lambda i: (0,) * len(shape))


def _rows(bsize, width):
    return pl.BlockSpec((bsize, width), lambda i: (i, 0))


def _enc_call(x, w1, b1, w2, b2, g, b, bsize, off_rows=0, m_out=None):
    m, f = x.shape
    m_out = m - off_rows if m_out is None else m_out
    off_b = off_rows // bsize
    in_spec = pl.BlockSpec((bsize, f), lambda i: (i + off_b, 0))
    return pl.pallas_call(
        _enc_body,
        grid=(m_out // bsize,),
        in_specs=[in_spec, _full(w1.shape), _full((1, LATENT)),
                  _full(w2.shape), _full((1, LATENT)), _full((1, LATENT)),
                  _full((1, LATENT))],
        out_specs=_rows(bsize, LATENT),
        out_shape=jax.ShapeDtypeStruct((m_out, LATENT), jnp.float32),
    )(x, w1, b1.reshape(1, -1), w2, b2.reshape(1, -1), g.reshape(1, -1),
      b.reshape(1, -1))


def _encn_call(x, w1, b1, w2, b2, g, b, wb, wc, bsize):
    m, f = x.shape
    return pl.pallas_call(
        _encn_body,
        grid=(m // bsize,),
        in_specs=[_rows(bsize, f), _full(w1.shape), _full((1, LATENT)),
                  _full(w2.shape), _full((1, LATENT)), _full((1, LATENT)),
                  _full((1, LATENT)), _full((LATENT, LATENT)),
                  _full((LATENT, LATENT))],
        out_specs=[_rows(bsize, LATENT)] * 3,
        out_shape=[jax.ShapeDtypeStruct((m, LATENT), jnp.float32)] * 3,
    )(x, w1, b1.reshape(1, -1), w2, b2.reshape(1, -1), g.reshape(1, -1),
      b.reshape(1, -1), wb, wc)


def _edge_call(e, gin, w1a, b1, w2, b2, g, b, bsize):
    m = e.shape[0]
    wspec = _full((LATENT, LATENT))
    vec = _full((1, LATENT))
    return pl.pallas_call(
        _edge_body,
        grid=(m // bsize,),
        in_specs=[_rows(bsize, LATENT)] * 2 + [wspec, vec,
                                               wspec, vec, vec, vec],
        out_specs=_rows(bsize, LATENT),
        out_shape=jax.ShapeDtypeStruct((m, LATENT), jnp.float32),
    )(e, gin, w1a, b1.reshape(1, -1), w2, b2.reshape(1, -1),
      g.reshape(1, -1), b.reshape(1, -1))


def _node_call(v, aggA, aggB, w1, b1, w2, b2, g, b, nwb, nwc, dec, bsize):
    wa, wb = w1[:LATENT], w1[LATENT:]
    wspec = _full((LATENT, LATENT))
    vec = _full((1, LATENT))
    agg_spec = pl.BlockSpec((NC, bsize, LATENT), lambda i: (0, i, 0))
    emit_u = nwb is not None
    in_specs = [_rows(bsize, LATENT), agg_spec, agg_spec, wspec, wspec, vec,
                wspec, vec, vec, vec]
    args = [v, aggA, aggB, wa, wb, b1.reshape(1, -1), w2, b2.reshape(1, -1),
            g.reshape(1, -1), b.reshape(1, -1)]
    if emit_u:
        in_specs += [wspec, wspec]
        args += [nwb, nwc]
        out_specs = [_rows(bsize, LATENT)] * 3
        out_shape = [jax.ShapeDtypeStruct((N, LATENT), jnp.float32)] * 3
    else:
        dw1, db1, dw2, db2 = dec
        out_f = dw2.shape[1]
        in_specs += [wspec, vec, _full((LATENT, out_f)), _full((1, out_f))]
        args += [dw1, db1.reshape(1, -1), dw2, db2.reshape(1, -1)]
        out_specs = _rows(bsize, out_f)
        out_shape = jax.ShapeDtypeStruct((N, out_f), jnp.float32)
    return pl.pallas_call(
        functools.partial(_node_body, emit_u),
        grid=(N // bsize,),
        in_specs=in_specs,
        out_specs=out_specs,
        out_shape=out_shape,
    )(*args)


def _dec_call(v, w1, b1, w2, b2, bsize):
    out_f = w2.shape[1]
    return pl.pallas_call(
        _dec_body,
        grid=(N // bsize,),
        in_specs=[_rows(bsize, LATENT), _full((LATENT, LATENT)),
                  _full((1, LATENT)), _full((LATENT, out_f)),
                  _full((1, out_f))],
        out_specs=_rows(bsize, out_f),
        out_shape=jax.ShapeDtypeStruct((N, out_f), jnp.float32),
    )(v, w1, b1.reshape(1, -1), w2, b2.reshape(1, -1))


def kernel(node_features, edge_features, params, senders, receivers):
    p = params
    w1 = [p['pe_W1'][t] for t in range(STEPS)]
    w1a = [w[:LATENT] for w in w1]
    w1b = [w[LATENT:2 * LATENT] for w in w1]
    w1c = [w[2 * LATENT:] for w in w1]

    v, ub, uc = _encn_call(node_features, p['enc_n_W1'], p['enc_n_b1'],
                           p['enc_n_W2'], p['enc_n_b2'], p['enc_n_g'],
                           p['enc_n_b'], w1b[0], w1c[0], 2000)
    enc_e_args = (p['enc_e_W1'], p['enc_e_b1'],
                  p['enc_e_W2'].astype(jnp.bfloat16),
                  p['enc_e_b2'], p['enc_e_g'], p['enc_e_b'])
    eA = _enc_call(edge_features, *enc_e_args, 4000, 0, EH)
    eB = _enc_call(edge_features, *enc_e_args, 4000, EH, EH)

    zeros = jnp.zeros((N_PAD, LATENT), jnp.float32)
    bf = jnp.bfloat16
    dec = (p['dec_W1'], p['dec_b1'], p['dec_W2'], p['dec_b2'])

    out = None
    for t in range(STEPS):
        ginA = _gatherA(ub, uc, senders, receivers)
        ginB = _gatherB(ub, uc, senders, receivers)
        edge_args = (w1a[t].astype(bf), p['pe_b1'][t],
                     p['pe_W2'][t].astype(bf), p['pe_b2'][t],
                     p['pe_g'][t], p['pe_b'][t], 4000)
        eA = _edge_call(eA, ginA, *edge_args)
        aggA = _segsumA(eA, receivers, zeros)
        eB = _edge_call(eB, ginB, *edge_args)
        aggB = _segsumB(eB, receivers, zeros)
        last = t == STEPS - 1
        res = _node_call(v, aggA, aggB, p['pn_W1'][t], p['pn_b1'][t],
                         p['pn_W2'][t], p['pn_b2'][t], p['pn_g'][t],
                         p['pn_b'][t],
                         None if last else w1b[t + 1],
                         None if last else w1c[t + 1],
                         dec if last else None, 2000)
        if last:
            out = res
        else:
            v, ub, uc = res

    return out
